# baseline (speedup 1.0000x reference)
import functools

import jax
import jax.numpy as jnp
from jax import lax
from jax.experimental import pallas as pl
from jax.experimental.pallas import tpu as pltpu
from jax.experimental.pallas import tpu_sc as plsc

_TT = 8


def _vq_body(x_ref, cbt_ref, idx_ref, *, K: int):
    for i in range(_TT):
        x = x_ref[i]
        cbt = cbt_ref[i]
        x_bf = x.astype(jnp.bfloat16)
        cbt_bf = cbt.astype(jnp.bfloat16)
        ab = jax.lax.dot_general(
            x_bf, cbt_bf, (((1,), (0,)), ((), ())),
            preferred_element_type=jnp.float32)
        a2 = jnp.sum(x * x, axis=1, keepdims=True)
        b2 = jnp.sum(cbt * cbt, axis=0, keepdims=True)
        scores = (a2 + b2) - 2.0 * ab
        idx = jnp.argmin(scores, axis=1).astype(jnp.int32)
        idx_ref[i, 0, :] = idx


def kernel(input, codebook):
    B, T, D = input.shape
    K = codebook.shape[1]
    x_t = jnp.moveaxis(input, 1, 0)
    cbt = jnp.swapaxes(codebook, 1, 2)
    idx_t = pl.pallas_call(
        functools.partial(_vq_body, K=K),
        grid=(T // _TT,),
        in_specs=[
            pl.BlockSpec((_TT, B, D), lambda t: (t, 0, 0)),
            pl.BlockSpec((_TT, D, K), lambda t: (t, 0, 0)),
        ],
        out_specs=pl.BlockSpec((_TT, 1, B), lambda t: (t, 0, 0)),
        out_shape=jax.ShapeDtypeStruct((T, 1, B), jnp.int32),
    )(x_t, cbt)
    idxes_tb = idx_t[:, 0, :]
    fidx = (idxes_tb
            + jnp.arange(T, dtype=jnp.int32)[:, None] * K).reshape(T * B)
    rows = _gather_sc(codebook.reshape(T * K, D), fidx)
    embed = jnp.moveaxis(rows.reshape(T, B, D), 0, 1)
    return embed, idxes_tb.T

def _gather_sc(table, fidx):
    N, D = table.shape
    M = fidx.shape[0]
    info = plsc.get_sparse_core_info()
    nw = info.num_cores * info.num_subcores
    m_per_w = M // nw
    mesh = plsc.VectorSubcoreMesh(core_axis_name="c", subcore_axis_name="s")

    @functools.partial(
        pl.kernel, mesh=mesh,
        out_type=jax.ShapeDtypeStruct((M, D), jnp.float32),
        compiler_params=pltpu.CompilerParams(use_tc_tiling_on_sc=False),
        scratch_types=[
            pltpu.VMEM((m_per_w,), jnp.int32),
            pltpu.VMEM((m_per_w, D), jnp.float32),
            pltpu.SemaphoreType.DMA,
        ],
    )
    def gather_kernel(table_hbm, fidx_hbm, out_hbm, idx_v, rows_v, sem):
        wid = lax.axis_index("s") * info.num_cores + lax.axis_index("c")
        base = wid * m_per_w
        pltpu.sync_copy(fidx_hbm.at[pl.ds(base, m_per_w)], idx_v)
        pltpu.async_copy(table_hbm.at[idx_v], rows_v, sem).wait()
        pltpu.sync_copy(rows_v, out_hbm.at[pl.ds(base, m_per_w)])

    return gather_kernel(table, fidx)


# speedup vs baseline: 6.3262x; 6.3262x over previous
import functools

import jax
import jax.numpy as jnp
from jax import lax
from jax.experimental import pallas as pl

_TT = 8


def _vq_body(x_ref, cbt_ref, idx_ref, emb_ref, *, K: int):
    for i in range(_TT):
        x = x_ref[i]
        cbt = cbt_ref[i]
        B = x.shape[0]
        x_bf = x.astype(jnp.bfloat16)
        cbt_bf = cbt.astype(jnp.bfloat16)
        ab = jax.lax.dot_general(
            x_bf, cbt_bf, (((1,), (0,)), ((), ())),
            preferred_element_type=jnp.float32)
        a2 = jnp.sum(x * x, axis=1, keepdims=True)
        b2 = jnp.sum(cbt * cbt, axis=0, keepdims=True)
        scores = (a2 + b2) - 2.0 * ab
        idx = jnp.argmin(scores, axis=1).astype(jnp.int32)
        idx_ref[i, 0, :] = idx
        kiota = lax.broadcasted_iota(jnp.int32, (B, K), 1)
        onehot = (kiota == idx[:, None]).astype(jnp.bfloat16)
        emb_ref[i] = jax.lax.dot_general(
            onehot, cbt_bf, (((1,), (1,)), ((), ())),
            preferred_element_type=jnp.float32)


def kernel(input, codebook):
    B, T, D = input.shape
    K = codebook.shape[1]
    x_t = jnp.moveaxis(input, 1, 0)
    cbt = jnp.swapaxes(codebook, 1, 2)
    idx_t, emb_t = pl.pallas_call(
        functools.partial(_vq_body, K=K),
        grid=(T // _TT,),
        in_specs=[
            pl.BlockSpec((_TT, B, D), lambda t: (t, 0, 0)),
            pl.BlockSpec((_TT, D, K), lambda t: (t, 0, 0)),
        ],
        out_specs=[
            pl.BlockSpec((_TT, 1, B), lambda t: (t, 0, 0)),
            pl.BlockSpec((_TT, B, D), lambda t: (t, 0, 0)),
        ],
        out_shape=[
            jax.ShapeDtypeStruct((T, 1, B), jnp.int32),
            jax.ShapeDtypeStruct((T, B, D), jnp.float32),
        ],
    )(x_t, cbt)
    embed = jnp.moveaxis(emb_t, 0, 1)
    return embed, idx_t[:, 0, :].T


# speedup vs baseline: 6.5909x; 1.0418x over previous
import functools

import jax
import jax.numpy as jnp
from jax import lax
from jax.experimental import pallas as pl

_TT = 16


def _vq_body(x_ref, cbt_ref, idx_ref, emb_ref, *, K: int):
    for i in range(_TT):
        x = x_ref[i]
        cbt = cbt_ref[i]
        B = x.shape[0]
        x_bf = x.astype(jnp.bfloat16)
        cbt_bf = cbt.astype(jnp.bfloat16)
        ab = jax.lax.dot_general(
            x_bf, cbt_bf, (((1,), (0,)), ((), ())),
            preferred_element_type=jnp.float32)
        a2 = jnp.sum(x * x, axis=1, keepdims=True)
        b2 = jnp.sum(cbt * cbt, axis=0, keepdims=True)
        scores = (a2 + b2) - 2.0 * ab
        idx = jnp.argmin(scores, axis=1).astype(jnp.int32)
        idx_ref[i, 0, :] = idx
        kiota = lax.broadcasted_iota(jnp.int32, (B, K), 1)
        onehot = (kiota == idx[:, None]).astype(jnp.bfloat16)
        emb_ref[i] = jax.lax.dot_general(
            onehot, cbt_bf, (((1,), (1,)), ((), ())),
            preferred_element_type=jnp.float32)


def kernel(input, codebook):
    B, T, D = input.shape
    K = codebook.shape[1]
    x_t = jnp.moveaxis(input, 1, 0)
    cbt = jnp.swapaxes(codebook, 1, 2)
    idx_t, emb_t = pl.pallas_call(
        functools.partial(_vq_body, K=K),
        grid=(T // _TT,),
        in_specs=[
            pl.BlockSpec((_TT, B, D), lambda t: (t, 0, 0)),
            pl.BlockSpec((_TT, D, K), lambda t: (t, 0, 0)),
        ],
        out_specs=[
            pl.BlockSpec((_TT, 1, B), lambda t: (t, 0, 0)),
            pl.BlockSpec((_TT, B, D), lambda t: (t, 0, 0)),
        ],
        out_shape=[
            jax.ShapeDtypeStruct((T, 1, B), jnp.int32),
            jax.ShapeDtypeStruct((T, B, D), jnp.float32),
        ],
    )(x_t, cbt)
    embed = jnp.moveaxis(emb_t, 0, 1)
    return embed, idx_t[:, 0, :].T
